# baseline (device time: 214094 ns/iter reference)
import jax
import jax.numpy as jnp
from jax import lax
from jax.experimental import pallas as pl
from jax.experimental.pallas import tpu as pltpu

E = 16
E_LOC = 8
T = 2048
T_LOC = 1024
D = 1024
F = 4096
CAP = 384
FT = 1024

_MESH = pl.DeviceIdType.MESH if hasattr(pl, "DeviceIdType") else pltpu.DeviceIdType.MESH
_CompilerParams = getattr(pltpu, "CompilerParams", None) or pltpu.TPUCompilerParams


def _partner():
    return (1 - lax.axis_index("x"), lax.axis_index("y"), lax.axis_index("z"))


def _partner_barrier():
    barrier = pltpu.get_barrier_semaphore()
    pl.semaphore_signal(barrier, inc=1, device_id=_partner(), device_id_type=_MESH)
    pl.semaphore_wait(barrier, 1)


def _pairwise_exchange(arrays, collective_id):
    n = len(arrays)

    def body(*refs):
        ins = refs[:n]
        outs = refs[n : 2 * n]
        send_sems = refs[2 * n]
        recv_sems = refs[2 * n + 1]
        _partner_barrier()
        rdmas = []
        for k in range(n):
            r = pltpu.make_async_remote_copy(
                src_ref=ins[k],
                dst_ref=outs[k],
                send_sem=send_sems.at[k],
                recv_sem=recv_sems.at[k],
                device_id=_partner(),
                device_id_type=_MESH,
            )
            r.start()
            rdmas.append(r)
        for r in rdmas:
            r.wait()

    out = pl.pallas_call(
        body,
        out_shape=[jax.ShapeDtypeStruct(a.shape, a.dtype) for a in arrays],
        in_specs=[pl.BlockSpec(memory_space=pltpu.VMEM)] * n,
        out_specs=[pl.BlockSpec(memory_space=pltpu.VMEM)] * n,
        scratch_shapes=[
            pltpu.SemaphoreType.DMA((n,)),
            pltpu.SemaphoreType.DMA((n,)),
        ],
        compiler_params=_CompilerParams(collective_id=collective_id),
    )(*arrays)
    return out


def _exchange_add(mine, theirs, collective_id):

    def body(mine_ref, theirs_ref, out_ref, recv_buf, send_sem, recv_sem):
        _partner_barrier()
        r = pltpu.make_async_remote_copy(
            src_ref=theirs_ref,
            dst_ref=recv_buf,
            send_sem=send_sem,
            recv_sem=recv_sem,
            device_id=_partner(),
            device_id_type=_MESH,
        )
        r.start()
        r.wait()
        out_ref[...] = mine_ref[...] + recv_buf[...].astype(mine_ref.dtype)

    return pl.pallas_call(
        body,
        out_shape=jax.ShapeDtypeStruct(mine.shape, mine.dtype),
        in_specs=[pl.BlockSpec(memory_space=pltpu.VMEM)] * 2,
        out_specs=pl.BlockSpec(memory_space=pltpu.VMEM),
        scratch_shapes=[
            pltpu.VMEM(theirs.shape, theirs.dtype),
            pltpu.SemaphoreType.DMA,
            pltpu.SemaphoreType.DMA,
        ],
        compiler_params=_CompilerParams(collective_id=collective_id),
    )(mine, theirs)


def _ffn(Xg, W1, W2):

    def body(xg_ref, w1_ref, w2_ref, yg_ref):
        @pl.when(pl.program_id(1) == 0)
        def _init():
            yg_ref[...] = jnp.zeros_like(yg_ref)

        h = jnp.dot(
            xg_ref[0],
            w1_ref[0].astype(jnp.bfloat16),
            preferred_element_type=jnp.float32,
        )
        h = jnp.maximum(h, 0.0).astype(jnp.bfloat16)
        yg_ref[0] += jnp.dot(
            h, w2_ref[0].astype(jnp.bfloat16), preferred_element_type=jnp.float32
        )

    return pl.pallas_call(
        body,
        grid=(E_LOC, F // FT),
        in_specs=[
            pl.BlockSpec((1, CAP, D), lambda e, f: (e, 0, 0)),
            pl.BlockSpec((1, D, FT), lambda e, f: (e, 0, f)),
            pl.BlockSpec((1, FT, D), lambda e, f: (e, f, 0)),
        ],
        out_specs=pl.BlockSpec((1, CAP, D), lambda e, f: (e, 0, 0)),
        out_shape=jax.ShapeDtypeStruct((E_LOC, CAP, D), jnp.float32),
        compiler_params=_CompilerParams(
            dimension_semantics=("arbitrary", "arbitrary")
        ),
    )(Xg, W1, W2)


def _ffn_fused(sel, x_full, W1, W2, Wg):
    n_ft = F // FT

    def body(sel_ref, xf_ref, w1_ref, w2_ref, wg_ref, out_ref, xg_scr, yg_scr):
        e = pl.program_id(0)
        f = pl.program_id(1)

        @pl.when(jnp.logical_and(e == 0, f == 0))
        def _init_out():
            out_ref[...] = jnp.zeros_like(out_ref)

        @pl.when(f == 0)
        def _dispatch():
            xg_scr[...] = jnp.dot(
                sel_ref[0], xf_ref[...], preferred_element_type=jnp.float32
            ).astype(jnp.bfloat16)

        h = jnp.dot(
            xg_scr[...],
            w1_ref[0].astype(jnp.bfloat16),
            preferred_element_type=jnp.float32,
        )
        h = jnp.maximum(h, 0.0).astype(jnp.bfloat16)
        yg = jnp.dot(
            h, w2_ref[0].astype(jnp.bfloat16), preferred_element_type=jnp.float32
        )

        @pl.when(f == 0)
        def _set():
            yg_scr[...] = yg

        @pl.when(f > 0)
        def _acc():
            yg_scr[...] += yg

        @pl.when(f == n_ft - 1)
        def _combine():
            contrib = (yg_scr[...] * wg_ref[0, 0][:, None]).astype(jnp.bfloat16)
            out_ref[...] += lax.dot_general(
                sel_ref[0],
                contrib,
                (((0,), (0,)), ((), ())),
                preferred_element_type=jnp.float32,
            )

    return pl.pallas_call(
        body,
        grid=(E_LOC, n_ft),
        in_specs=[
            pl.BlockSpec((1, CAP, T), lambda e, f: (e, 0, 0)),
            pl.BlockSpec((T, D), lambda e, f: (0, 0)),
            pl.BlockSpec((1, D, FT), lambda e, f: (e, 0, f)),
            pl.BlockSpec((1, FT, D), lambda e, f: (e, f, 0)),
            pl.BlockSpec((1, 1, CAP), lambda e, f: (e, 0, 0)),
        ],
        out_specs=pl.BlockSpec((T, D), lambda e, f: (0, 0)),
        out_shape=jax.ShapeDtypeStruct((T, D), jnp.float32),
        scratch_shapes=[
            pltpu.VMEM((CAP, D), jnp.bfloat16),
            pltpu.VMEM((CAP, D), jnp.float32),
        ],
        compiler_params=_CompilerParams(
            dimension_semantics=("arbitrary", "arbitrary")
        ),
    )(sel, x_full, W1, W2, Wg.reshape(E_LOC, 1, CAP))


def kernel(x, router, W1, W2):
    my_x = lax.axis_index("x")

    def cat(a, b, axis):
        return jnp.where(
            my_x == 0,
            jnp.concatenate([a, b], axis=axis),
            jnp.concatenate([b, a], axis=axis),
        )

    x_bf = x.astype(jnp.bfloat16)
    router_other, x_other = _pairwise_exchange([router, x_bf], collective_id=0)
    router_full = cat(router, router_other, axis=1)
    x_full = cat(x_bf, x_other, axis=0)

    gates_loc = jnp.dot(x, router_full, precision=lax.Precision.HIGHEST)
    (gates_other,) = _pairwise_exchange([gates_loc], collective_id=1)
    gates = cat(gates_loc, gates_other, axis=0)

    top_v, top_i = lax.top_k(gates, 2)
    w = jnp.exp(top_v - top_v[:, 0:1])
    w = w / w.sum(axis=1, keepdims=True)
    onehot = top_i[:, :, None] == jnp.arange(E)[None, None, :]
    w_dense = jnp.sum(onehot * w[:, :, None], axis=1)
    w_loc = lax.dynamic_slice(w_dense, (0, my_x * E_LOC), (T, E_LOC))

    chosen = w_loc > 0
    order = jnp.argsort(~chosen, axis=0, stable=True)
    idx = order[:CAP, :].T
    sel = (idx[:, :, None] == jnp.arange(T)[None, None, :]).astype(jnp.bfloat16)
    Wg = jnp.einsum("ect,et->ec", sel.astype(jnp.float32), w_loc.T)

    partial = _ffn_fused(sel, x_full, W1, W2, Wg)

    mine = lax.dynamic_slice(partial, (my_x * T_LOC, 0), (T_LOC, D))
    theirs = lax.dynamic_slice(partial, ((1 - my_x) * T_LOC, 0), (T_LOC, D))
    return _exchange_add(mine, theirs.astype(jnp.bfloat16), collective_id=2)


# device time: 205697 ns/iter; 1.0408x vs baseline; 1.0408x over previous
import jax
import jax.numpy as jnp
from jax import lax
from jax.experimental import pallas as pl
from jax.experimental.pallas import tpu as pltpu

E = 16
E_LOC = 8
T = 2048
T_LOC = 1024
D = 1024
F = 4096
CAP = 320
FT = 1024

_MESH = pl.DeviceIdType.MESH if hasattr(pl, "DeviceIdType") else pltpu.DeviceIdType.MESH
_CompilerParams = getattr(pltpu, "CompilerParams", None) or pltpu.TPUCompilerParams


def _partner():
    return (1 - lax.axis_index("x"), lax.axis_index("y"), lax.axis_index("z"))


def _partner_barrier():
    barrier = pltpu.get_barrier_semaphore()
    pl.semaphore_signal(barrier, inc=1, device_id=_partner(), device_id_type=_MESH)
    pl.semaphore_wait(barrier, 1)


def _pairwise_exchange(arrays, collective_id):
    n = len(arrays)

    def body(*refs):
        ins = refs[:n]
        outs = refs[n : 2 * n]
        send_sems = refs[2 * n]
        recv_sems = refs[2 * n + 1]
        _partner_barrier()
        rdmas = []
        for k in range(n):
            r = pltpu.make_async_remote_copy(
                src_ref=ins[k],
                dst_ref=outs[k],
                send_sem=send_sems.at[k],
                recv_sem=recv_sems.at[k],
                device_id=_partner(),
                device_id_type=_MESH,
            )
            r.start()
            rdmas.append(r)
        for r in rdmas:
            r.wait()

    out = pl.pallas_call(
        body,
        out_shape=[jax.ShapeDtypeStruct(a.shape, a.dtype) for a in arrays],
        in_specs=[pl.BlockSpec(memory_space=pltpu.VMEM)] * n,
        out_specs=[pl.BlockSpec(memory_space=pltpu.VMEM)] * n,
        scratch_shapes=[
            pltpu.SemaphoreType.DMA((n,)),
            pltpu.SemaphoreType.DMA((n,)),
        ],
        compiler_params=_CompilerParams(collective_id=collective_id),
    )(*arrays)
    return out


def _exchange_add(mine, theirs, collective_id):

    def body(mine_ref, theirs_ref, out_ref, recv_buf, send_sem, recv_sem):
        _partner_barrier()
        r = pltpu.make_async_remote_copy(
            src_ref=theirs_ref,
            dst_ref=recv_buf,
            send_sem=send_sem,
            recv_sem=recv_sem,
            device_id=_partner(),
            device_id_type=_MESH,
        )
        r.start()
        r.wait()
        out_ref[...] = mine_ref[...] + recv_buf[...].astype(mine_ref.dtype)

    return pl.pallas_call(
        body,
        out_shape=jax.ShapeDtypeStruct(mine.shape, mine.dtype),
        in_specs=[pl.BlockSpec(memory_space=pltpu.VMEM)] * 2,
        out_specs=pl.BlockSpec(memory_space=pltpu.VMEM),
        scratch_shapes=[
            pltpu.VMEM(theirs.shape, theirs.dtype),
            pltpu.SemaphoreType.DMA,
            pltpu.SemaphoreType.DMA,
        ],
        compiler_params=_CompilerParams(collective_id=collective_id),
    )(mine, theirs)


def _ffn(Xg, W1, W2):

    def body(xg_ref, w1_ref, w2_ref, yg_ref):
        @pl.when(pl.program_id(1) == 0)
        def _init():
            yg_ref[...] = jnp.zeros_like(yg_ref)

        h = jnp.dot(
            xg_ref[0],
            w1_ref[0].astype(jnp.bfloat16),
            preferred_element_type=jnp.float32,
        )
        h = jnp.maximum(h, 0.0).astype(jnp.bfloat16)
        yg_ref[0] += jnp.dot(
            h, w2_ref[0].astype(jnp.bfloat16), preferred_element_type=jnp.float32
        )

    return pl.pallas_call(
        body,
        grid=(E_LOC, F // FT),
        in_specs=[
            pl.BlockSpec((1, CAP, D), lambda e, f: (e, 0, 0)),
            pl.BlockSpec((1, D, FT), lambda e, f: (e, 0, f)),
            pl.BlockSpec((1, FT, D), lambda e, f: (e, f, 0)),
        ],
        out_specs=pl.BlockSpec((1, CAP, D), lambda e, f: (e, 0, 0)),
        out_shape=jax.ShapeDtypeStruct((E_LOC, CAP, D), jnp.float32),
        compiler_params=_CompilerParams(
            dimension_semantics=("arbitrary", "arbitrary")
        ),
    )(Xg, W1, W2)


def _ffn_fused(sel, x_full, W1, W2, Wg):
    n_ft = F // FT

    def body(sel_ref, xf_ref, w1_ref, w2_ref, wg_ref, out_ref, xg_scr, yg_scr):
        e = pl.program_id(0)
        f = pl.program_id(1)

        @pl.when(jnp.logical_and(e == 0, f == 0))
        def _init_out():
            out_ref[...] = jnp.zeros_like(out_ref)

        @pl.when(f == 0)
        def _dispatch():
            xg_scr[...] = jnp.dot(
                sel_ref[0], xf_ref[...], preferred_element_type=jnp.float32
            ).astype(jnp.bfloat16)

        h = jnp.dot(
            xg_scr[...],
            w1_ref[0].astype(jnp.bfloat16),
            preferred_element_type=jnp.float32,
        )
        h = jnp.maximum(h, 0.0).astype(jnp.bfloat16)
        yg = jnp.dot(
            h, w2_ref[0].astype(jnp.bfloat16), preferred_element_type=jnp.float32
        )

        @pl.when(f == 0)
        def _set():
            yg_scr[...] = yg

        @pl.when(f > 0)
        def _acc():
            yg_scr[...] += yg

        @pl.when(f == n_ft - 1)
        def _combine():
            contrib = (yg_scr[...] * wg_ref[0, 0][:, None]).astype(jnp.bfloat16)
            out_ref[...] += lax.dot_general(
                sel_ref[0],
                contrib,
                (((0,), (0,)), ((), ())),
                preferred_element_type=jnp.float32,
            )

    return pl.pallas_call(
        body,
        grid=(E_LOC, n_ft),
        in_specs=[
            pl.BlockSpec((1, CAP, T), lambda e, f: (e, 0, 0)),
            pl.BlockSpec((T, D), lambda e, f: (0, 0)),
            pl.BlockSpec((1, D, FT), lambda e, f: (e, 0, f)),
            pl.BlockSpec((1, FT, D), lambda e, f: (e, f, 0)),
            pl.BlockSpec((1, 1, CAP), lambda e, f: (e, 0, 0)),
        ],
        out_specs=pl.BlockSpec((T, D), lambda e, f: (0, 0)),
        out_shape=jax.ShapeDtypeStruct((T, D), jnp.float32),
        scratch_shapes=[
            pltpu.VMEM((CAP, D), jnp.bfloat16),
            pltpu.VMEM((CAP, D), jnp.float32),
        ],
        compiler_params=_CompilerParams(
            dimension_semantics=("arbitrary", "arbitrary")
        ),
    )(sel, x_full, W1, W2, Wg.reshape(E_LOC, 1, CAP))


def kernel(x, router, W1, W2):
    my_x = lax.axis_index("x")

    def cat(a, b, axis):
        return jnp.where(
            my_x == 0,
            jnp.concatenate([a, b], axis=axis),
            jnp.concatenate([b, a], axis=axis),
        )

    x_bf = x.astype(jnp.bfloat16)
    router_other, x_other = _pairwise_exchange([router, x_bf], collective_id=0)
    router_full = cat(router, router_other, axis=1)
    x_full = cat(x_bf, x_other, axis=0)

    gates_loc = jnp.dot(x, router_full, precision=lax.Precision.HIGHEST)
    (gates_other,) = _pairwise_exchange([gates_loc], collective_id=1)
    gates = cat(gates_loc, gates_other, axis=0)

    top_v, top_i = lax.top_k(gates, 2)
    w = jnp.exp(top_v - top_v[:, 0:1])
    w = w / w.sum(axis=1, keepdims=True)
    onehot = top_i[:, :, None] == jnp.arange(E)[None, None, :]
    w_dense = jnp.sum(onehot * w[:, :, None], axis=1)
    w_loc = lax.dynamic_slice(w_dense, (0, my_x * E_LOC), (T, E_LOC))

    chosen = w_loc > 0
    order = jnp.argsort(~chosen, axis=0, stable=True)
    idx = order[:CAP, :].T
    sel = (idx[:, :, None] == jnp.arange(T)[None, None, :]).astype(jnp.bfloat16)
    Wg = jnp.einsum("ect,et->ec", sel.astype(jnp.float32), w_loc.T)

    partial = _ffn_fused(sel, x_full, W1, W2, Wg)

    mine = lax.dynamic_slice(partial, (my_x * T_LOC, 0), (T_LOC, D))
    theirs = lax.dynamic_slice(partial, ((1 - my_x) * T_LOC, 0), (T_LOC, D))
    return _exchange_add(mine, theirs.astype(jnp.bfloat16), collective_id=2)


# device time: 200487 ns/iter; 1.0679x vs baseline; 1.0260x over previous
import jax
import jax.numpy as jnp
from jax import lax
from jax.experimental import pallas as pl
from jax.experimental.pallas import tpu as pltpu

E = 16
E_LOC = 8
T = 2048
T_LOC = 1024
D = 1024
F = 4096
CAP = 320
FT = 1024

_MESH = pl.DeviceIdType.MESH if hasattr(pl, "DeviceIdType") else pltpu.DeviceIdType.MESH
_CompilerParams = getattr(pltpu, "CompilerParams", None) or pltpu.TPUCompilerParams


def _partner():
    return (1 - lax.axis_index("x"), lax.axis_index("y"), lax.axis_index("z"))


def _partner_barrier():
    barrier = pltpu.get_barrier_semaphore()
    pl.semaphore_signal(barrier, inc=1, device_id=_partner(), device_id_type=_MESH)
    pl.semaphore_wait(barrier, 1)


def _pairwise_exchange(arrays, collective_id):
    n = len(arrays)

    def body(*refs):
        ins = refs[:n]
        outs = refs[n : 2 * n]
        send_sems = refs[2 * n]
        recv_sems = refs[2 * n + 1]
        _partner_barrier()
        rdmas = []
        for k in range(n):
            r = pltpu.make_async_remote_copy(
                src_ref=ins[k],
                dst_ref=outs[k],
                send_sem=send_sems.at[k],
                recv_sem=recv_sems.at[k],
                device_id=_partner(),
                device_id_type=_MESH,
            )
            r.start()
            rdmas.append(r)
        for r in rdmas:
            r.wait()

    out = pl.pallas_call(
        body,
        out_shape=[jax.ShapeDtypeStruct(a.shape, a.dtype) for a in arrays],
        in_specs=[pl.BlockSpec(memory_space=pltpu.VMEM)] * n,
        out_specs=[pl.BlockSpec(memory_space=pltpu.VMEM)] * n,
        scratch_shapes=[
            pltpu.SemaphoreType.DMA((n,)),
            pltpu.SemaphoreType.DMA((n,)),
        ],
        compiler_params=_CompilerParams(collective_id=collective_id),
    )(*arrays)
    return out


def _exchange_add(mine, theirs, collective_id):

    def body(mine_ref, theirs_ref, out_ref, recv_buf, send_sem, recv_sem):
        _partner_barrier()
        r = pltpu.make_async_remote_copy(
            src_ref=theirs_ref,
            dst_ref=recv_buf,
            send_sem=send_sem,
            recv_sem=recv_sem,
            device_id=_partner(),
            device_id_type=_MESH,
        )
        r.start()
        r.wait()
        out_ref[...] = mine_ref[...] + recv_buf[...].astype(mine_ref.dtype)

    return pl.pallas_call(
        body,
        out_shape=jax.ShapeDtypeStruct(mine.shape, mine.dtype),
        in_specs=[pl.BlockSpec(memory_space=pltpu.VMEM)] * 2,
        out_specs=pl.BlockSpec(memory_space=pltpu.VMEM),
        scratch_shapes=[
            pltpu.VMEM(theirs.shape, theirs.dtype),
            pltpu.SemaphoreType.DMA,
            pltpu.SemaphoreType.DMA,
        ],
        compiler_params=_CompilerParams(collective_id=collective_id),
    )(mine, theirs)


def _ffn(Xg, W1, W2):

    def body(xg_ref, w1_ref, w2_ref, yg_ref):
        @pl.when(pl.program_id(1) == 0)
        def _init():
            yg_ref[...] = jnp.zeros_like(yg_ref)

        h = jnp.dot(
            xg_ref[0],
            w1_ref[0].astype(jnp.bfloat16),
            preferred_element_type=jnp.float32,
        )
        h = jnp.maximum(h, 0.0).astype(jnp.bfloat16)
        yg_ref[0] += jnp.dot(
            h, w2_ref[0].astype(jnp.bfloat16), preferred_element_type=jnp.float32
        )

    return pl.pallas_call(
        body,
        grid=(E_LOC, F // FT),
        in_specs=[
            pl.BlockSpec((1, CAP, D), lambda e, f: (e, 0, 0)),
            pl.BlockSpec((1, D, FT), lambda e, f: (e, 0, f)),
            pl.BlockSpec((1, FT, D), lambda e, f: (e, f, 0)),
        ],
        out_specs=pl.BlockSpec((1, CAP, D), lambda e, f: (e, 0, 0)),
        out_shape=jax.ShapeDtypeStruct((E_LOC, CAP, D), jnp.float32),
        compiler_params=_CompilerParams(
            dimension_semantics=("arbitrary", "arbitrary")
        ),
    )(Xg, W1, W2)


def _ffn_fused(idx, x_full, W1, W2, Wg):
    n_ft = F // FT

    def body(idx_ref, xf_ref, w1_ref, w2_ref, wg_ref, out_ref,
             sel_scr, xg_scr, yg_scr):
        e = pl.program_id(0)
        f = pl.program_id(1)

        @pl.when(jnp.logical_and(e == 0, f == 0))
        def _init_out():
            out_ref[...] = jnp.zeros_like(out_ref)

        @pl.when(f == 0)
        def _dispatch():
            iot = lax.broadcasted_iota(jnp.int32, (CAP, T), 1)
            sel_scr[...] = (idx_ref[0, 0][:, None] == iot).astype(jnp.bfloat16)
            xg_scr[...] = jnp.dot(
                sel_scr[...], xf_ref[...], preferred_element_type=jnp.float32
            ).astype(jnp.bfloat16)

        h = jnp.dot(
            xg_scr[...],
            w1_ref[0].astype(jnp.bfloat16),
            preferred_element_type=jnp.float32,
        )
        h = jnp.maximum(h, 0.0).astype(jnp.bfloat16)
        yg = jnp.dot(
            h, w2_ref[0].astype(jnp.bfloat16), preferred_element_type=jnp.float32
        )

        @pl.when(f == 0)
        def _set():
            yg_scr[...] = yg

        @pl.when(f > 0)
        def _acc():
            yg_scr[...] += yg

        @pl.when(f == n_ft - 1)
        def _combine():
            contrib = (yg_scr[...] * wg_ref[0, 0][:, None]).astype(jnp.bfloat16)
            out_ref[...] += lax.dot_general(
                sel_scr[...],
                contrib,
                (((0,), (0,)), ((), ())),
                preferred_element_type=jnp.float32,
            )

    return pl.pallas_call(
        body,
        grid=(E_LOC, n_ft),
        in_specs=[
            pl.BlockSpec((1, 1, CAP), lambda e, f: (e, 0, 0)),
            pl.BlockSpec((T, D), lambda e, f: (0, 0)),
            pl.BlockSpec((1, D, FT), lambda e, f: (e, 0, f)),
            pl.BlockSpec((1, FT, D), lambda e, f: (e, f, 0)),
            pl.BlockSpec((1, 1, CAP), lambda e, f: (e, 0, 0)),
        ],
        out_specs=pl.BlockSpec((T, D), lambda e, f: (0, 0)),
        out_shape=jax.ShapeDtypeStruct((T, D), jnp.float32),
        scratch_shapes=[
            pltpu.VMEM((CAP, T), jnp.bfloat16),
            pltpu.VMEM((CAP, D), jnp.bfloat16),
            pltpu.VMEM((CAP, D), jnp.float32),
        ],
        compiler_params=_CompilerParams(
            dimension_semantics=("arbitrary", "arbitrary")
        ),
    )(idx.reshape(E_LOC, 1, CAP), x_full, W1, W2, Wg.reshape(E_LOC, 1, CAP))


def kernel(x, router, W1, W2):
    my_x = lax.axis_index("x")

    def cat(a, b, axis):
        return jnp.where(
            my_x == 0,
            jnp.concatenate([a, b], axis=axis),
            jnp.concatenate([b, a], axis=axis),
        )

    x_bf = x.astype(jnp.bfloat16)
    router_other, x_other = _pairwise_exchange([router, x_bf], collective_id=0)
    router_full = cat(router, router_other, axis=1)
    x_full = cat(x_bf, x_other, axis=0)

    gates_loc = jnp.dot(x, router_full, precision=lax.Precision.HIGHEST)
    (gates_other,) = _pairwise_exchange([gates_loc], collective_id=1)
    gates = cat(gates_loc, gates_other, axis=0)

    top_v, top_i = lax.top_k(gates, 2)
    w = jnp.exp(top_v - top_v[:, 0:1])
    w = w / w.sum(axis=1, keepdims=True)
    onehot = top_i[:, :, None] == jnp.arange(E)[None, None, :]
    w_dense = jnp.sum(onehot * w[:, :, None], axis=1)
    w_loc = lax.dynamic_slice(w_dense, (0, my_x * E_LOC), (T, E_LOC))

    keys = (w_loc <= 0).astype(jnp.int32)
    tok = lax.broadcasted_iota(jnp.int32, (T, E_LOC), 0)
    _, order, wsort = lax.sort(
        (keys, tok, w_loc), dimension=0, num_keys=1, is_stable=True
    )
    idx = order[:CAP].T
    Wg = wsort[:CAP].T

    partial = _ffn_fused(idx, x_full, W1, W2, Wg)

    mine = lax.dynamic_slice(partial, (my_x * T_LOC, 0), (T_LOC, D))
    theirs = lax.dynamic_slice(partial, ((1 - my_x) * T_LOC, 0), (T_LOC, D))
    return _exchange_add(mine, theirs.astype(jnp.bfloat16), collective_id=2)


# device time: 199947 ns/iter; 1.0708x vs baseline; 1.0027x over previous
import jax
import jax.numpy as jnp
from jax import lax
from jax.experimental import pallas as pl
from jax.experimental.pallas import tpu as pltpu

E = 16
E_LOC = 8
T = 2048
T_LOC = 1024
D = 1024
F = 4096
CAP = 320
FT = 1024

_MESH = pl.DeviceIdType.MESH if hasattr(pl, "DeviceIdType") else pltpu.DeviceIdType.MESH
_CompilerParams = getattr(pltpu, "CompilerParams", None) or pltpu.TPUCompilerParams


def _partner():
    return (1 - lax.axis_index("x"), lax.axis_index("y"), lax.axis_index("z"))


def _partner_barrier():
    barrier = pltpu.get_barrier_semaphore()
    pl.semaphore_signal(barrier, inc=1, device_id=_partner(), device_id_type=_MESH)
    pl.semaphore_wait(barrier, 1)


def _pairwise_exchange(arrays, collective_id):
    n = len(arrays)

    def body(*refs):
        ins = refs[:n]
        outs = refs[n : 2 * n]
        send_sems = refs[2 * n]
        recv_sems = refs[2 * n + 1]
        _partner_barrier()
        rdmas = []
        for k in range(n):
            r = pltpu.make_async_remote_copy(
                src_ref=ins[k],
                dst_ref=outs[k],
                send_sem=send_sems.at[k],
                recv_sem=recv_sems.at[k],
                device_id=_partner(),
                device_id_type=_MESH,
            )
            r.start()
            rdmas.append(r)
        for r in rdmas:
            r.wait()

    out = pl.pallas_call(
        body,
        out_shape=[jax.ShapeDtypeStruct(a.shape, a.dtype) for a in arrays],
        in_specs=[pl.BlockSpec(memory_space=pltpu.VMEM)] * n,
        out_specs=[pl.BlockSpec(memory_space=pltpu.VMEM)] * n,
        scratch_shapes=[
            pltpu.SemaphoreType.DMA((n,)),
            pltpu.SemaphoreType.DMA((n,)),
        ],
        compiler_params=_CompilerParams(collective_id=collective_id),
    )(*arrays)
    return out


def _exchange_add(mine, theirs, collective_id):

    def body(mine_ref, theirs_ref, out_ref, recv_buf, send_sem, recv_sem):
        _partner_barrier()
        r = pltpu.make_async_remote_copy(
            src_ref=theirs_ref,
            dst_ref=recv_buf,
            send_sem=send_sem,
            recv_sem=recv_sem,
            device_id=_partner(),
            device_id_type=_MESH,
        )
        r.start()
        r.wait()
        out_ref[...] = mine_ref[...] + recv_buf[...].astype(mine_ref.dtype)

    return pl.pallas_call(
        body,
        out_shape=jax.ShapeDtypeStruct(mine.shape, mine.dtype),
        in_specs=[pl.BlockSpec(memory_space=pltpu.VMEM)] * 2,
        out_specs=pl.BlockSpec(memory_space=pltpu.VMEM),
        scratch_shapes=[
            pltpu.VMEM(theirs.shape, theirs.dtype),
            pltpu.SemaphoreType.DMA,
            pltpu.SemaphoreType.DMA,
        ],
        compiler_params=_CompilerParams(collective_id=collective_id),
    )(mine, theirs)


def _ffn_fused(idx, x_full, W1, W2, Wg):
    n_ft = F // FT

    def body(idx_ref, xf_ref, w1_ref, w2_ref, wg_ref, out_ref,
             sel_scr, xg_scr, yg_scr):
        e = pl.program_id(0)
        f = pl.program_id(1)

        @pl.when(jnp.logical_and(e == 0, f == 0))
        def _init_out():
            out_ref[...] = jnp.zeros_like(out_ref)

        @pl.when(f == 0)
        def _dispatch():
            iot = lax.broadcasted_iota(jnp.int32, (CAP, T), 1)
            sel_scr[...] = (idx_ref[0, 0][:, None] == iot).astype(jnp.bfloat16)
            xg_scr[...] = jnp.dot(
                sel_scr[...], xf_ref[...], preferred_element_type=jnp.float32
            ).astype(jnp.bfloat16)

        h = jnp.dot(
            xg_scr[...],
            w1_ref[0].astype(jnp.bfloat16),
            preferred_element_type=jnp.float32,
        )
        h = jnp.maximum(h, 0.0).astype(jnp.bfloat16)
        yg = jnp.dot(
            h, w2_ref[0].astype(jnp.bfloat16), preferred_element_type=jnp.float32
        )

        @pl.when(f == 0)
        def _set():
            yg_scr[...] = yg

        @pl.when(f > 0)
        def _acc():
            yg_scr[...] += yg

        @pl.when(f == n_ft - 1)
        def _combine():
            contrib = (yg_scr[...] * wg_ref[0, 0][:, None]).astype(jnp.bfloat16)
            out_ref[...] += lax.dot_general(
                sel_scr[...],
                contrib,
                (((0,), (0,)), ((), ())),
                preferred_element_type=jnp.float32,
            )

    return pl.pallas_call(
        body,
        grid=(E_LOC, n_ft),
        in_specs=[
            pl.BlockSpec((1, 1, CAP), lambda e, f: (e, 0, 0)),
            pl.BlockSpec((T, D), lambda e, f: (0, 0)),
            pl.BlockSpec((1, D, FT), lambda e, f: (e, 0, f)),
            pl.BlockSpec((1, FT, D), lambda e, f: (e, f, 0)),
            pl.BlockSpec((1, 1, CAP), lambda e, f: (e, 0, 0)),
        ],
        out_specs=pl.BlockSpec((T, D), lambda e, f: (0, 0)),
        out_shape=jax.ShapeDtypeStruct((T, D), jnp.float32),
        scratch_shapes=[
            pltpu.VMEM((CAP, T), jnp.bfloat16),
            pltpu.VMEM((CAP, D), jnp.bfloat16),
            pltpu.VMEM((CAP, D), jnp.float32),
        ],
        compiler_params=_CompilerParams(
            dimension_semantics=("arbitrary", "arbitrary")
        ),
    )(idx.reshape(E_LOC, 1, CAP), x_full, W1, W2, Wg.reshape(E_LOC, 1, CAP))


def kernel(x, router, W1, W2):
    my_x = lax.axis_index("x")

    def cat(a, b, axis):
        return jnp.where(
            my_x == 0,
            jnp.concatenate([a, b], axis=axis),
            jnp.concatenate([b, a], axis=axis),
        )

    x_bf = x.astype(jnp.bfloat16)
    router_other, x_other = _pairwise_exchange([router, x_bf], collective_id=0)
    router_full = cat(router, router_other, axis=1)
    x_full = cat(x_bf, x_other, axis=0)

    gates_loc = jnp.dot(x, router_full, precision=lax.Precision.HIGHEST)
    (gates_other,) = _pairwise_exchange([gates_loc], collective_id=1)
    gates = cat(gates_loc, gates_other, axis=0)

    top_v, top_i = lax.top_k(gates, 2)
    w = jnp.exp(top_v - top_v[:, 0:1])
    w = w / w.sum(axis=1, keepdims=True)
    onehot = top_i[:, :, None] == jnp.arange(E)[None, None, :]
    w_dense = jnp.sum(onehot * w[:, :, None], axis=1)
    w_loc = lax.dynamic_slice(w_dense, (0, my_x * E_LOC), (T, E_LOC))

    keys = (w_loc <= 0).astype(jnp.int32)
    tok = lax.broadcasted_iota(jnp.int32, (T, E_LOC), 0)
    _, order, wsort = lax.sort(
        (keys, tok, w_loc), dimension=0, num_keys=1, is_stable=True
    )
    idx = order[:CAP].T
    Wg = wsort[:CAP].T

    partial = _ffn_fused(idx, x_full, W1, W2, Wg)

    mine = lax.dynamic_slice(partial, (my_x * T_LOC, 0), (T_LOC, D))
    theirs = lax.dynamic_slice(partial, ((1 - my_x) * T_LOC, 0), (T_LOC, D))
    return _exchange_add(mine, theirs.astype(jnp.bfloat16), collective_id=2)
